# Initial kernel scaffold; baseline (speedup 1.0000x reference)
import jax
import jax.numpy as jnp
from jax.experimental import pallas as pl


def kernel(x, w1, w2, w3, cw1, cw2, cw3, g1, b1, g2, b2, g3, b3):
    raise NotImplementedError("write your pallas kernel here")


# speedup vs baseline: 1.9016x; 1.9016x over previous
from functools import partial

import jax
import jax.numpy as jnp
from jax.experimental import pallas as pl
from jax.experimental.pallas import tpu as pltpu

_EPS = 1e-5
_VMEM_LIMIT = 64 * 1024 * 1024
_TILE = 2048


def _params():
    return pltpu.CompilerParams(
        dimension_semantics=("parallel", "parallel"),
        vmem_limit_bytes=_VMEM_LIMIT,
    )


def _const_spec(arr):
    return pl.BlockSpec(arr.shape, lambda n, t: (0,) * arr.ndim)


def _stats(h, sum_ref, sq_ref):
    sum_ref[...] = jnp.sum(h, axis=1)[None, :]
    sq_ref[...] = jnp.sum(h * h, axis=1)[None, :]


def _pass1_kernel(x_ref, w1_ref, h1_ref, sum_ref, sq_ref):
    xb = x_ref[...].astype(jnp.bfloat16)
    h = jnp.dot(w1_ref[...], xb, preferred_element_type=jnp.float32)
    _stats(h, sum_ref, sq_ref)
    h1_ref[...] = h.astype(jnp.bfloat16)


def _pass2_kernel(h1_ref, w2_ref, s1_ref, t1_ref, h2_ref, sum_ref, sq_ref):
    h1 = h1_ref[...].astype(jnp.float32)
    a1 = jnp.maximum(h1 * s1_ref[...] + t1_ref[...], 0.0).astype(jnp.bfloat16)
    h = jnp.dot(w2_ref[...], a1, preferred_element_type=jnp.float32)
    _stats(h, sum_ref, sq_ref)
    h2_ref[...] = h.astype(jnp.bfloat16)


def _pass3_kernel(h2_ref, w3_ref, s2_ref, t2_ref, a2_ref, sum_ref, sq_ref):
    h2 = h2_ref[...].astype(jnp.float32)
    a2 = jnp.maximum(h2 * s2_ref[...] + t2_ref[...], 0.0).astype(jnp.bfloat16)
    h = jnp.dot(w3_ref[...], a2, preferred_element_type=jnp.float32)
    _stats(h, sum_ref, sq_ref)
    a2_ref[...] = a2


def _pass4_kernel(x_ref, a2_ref, w3_ref, s3_ref, t3_ref, o_ref):
    h = jnp.dot(w3_ref[...], a2_ref[...], preferred_element_type=jnp.float32)
    h = h * s3_ref[...] + t3_ref[...]
    o_ref[...] = jnp.maximum(x_ref[...].astype(jnp.float32) + h, 0.0).astype(
        o_ref.dtype)


def _bn_fold(sums, sqs, gamma, beta, count):
    s = jnp.sum(sums, axis=(0, 1, 2))
    ss = jnp.sum(sqs, axis=(0, 1, 2))
    mean = s / count
    var = ss / count - mean * mean
    inv = jax.lax.rsqrt(var + _EPS)
    scale = gamma * inv
    shift = beta - mean * scale
    return scale.reshape(-1, 1), shift.reshape(-1, 1)


def kernel(x, w1, w2, w3, cw1, cw2, cw3, g1, b1, g2, b2, g3, b3):
    N, Cin, H, W = x.shape
    c4 = w1.shape[0]
    Cout = w3.shape[0]
    HW = H * W
    tile = _TILE if HW % _TILE == 0 else HW
    T = HW // tile
    count = N * HW

    x3 = x.reshape(N, Cin, HW)
    w1b = w1.astype(jnp.bfloat16)
    w2b = w2.astype(jnp.bfloat16)
    w3b = w3.astype(jnp.bfloat16)

    stats_sd = lambda c: jax.ShapeDtypeStruct((N, T, 1, c), jnp.float32)
    stats_spec = lambda c: pl.BlockSpec((None, None, 1, c),
                                        lambda n, t: (n, t, 0, 0))

    h1, s1p, q1p = pl.pallas_call(
        _pass1_kernel,
        out_shape=(jax.ShapeDtypeStruct((N, c4, HW), jnp.bfloat16),
                   stats_sd(c4), stats_sd(c4)),
        grid=(N, T),
        in_specs=[pl.BlockSpec((None, Cin, tile), lambda n, t: (n, 0, t)),
                  _const_spec(w1b)],
        out_specs=(pl.BlockSpec((None, c4, tile), lambda n, t: (n, 0, t)),
                   stats_spec(c4), stats_spec(c4)),
        compiler_params=_params(),
    )(x3, w1b)
    s1, t1 = _bn_fold(s1p, q1p, g1, b1, count)

    h2, s2p, q2p = pl.pallas_call(
        _pass2_kernel,
        out_shape=(jax.ShapeDtypeStruct((N, c4, HW), jnp.bfloat16),
                   stats_sd(c4), stats_sd(c4)),
        grid=(N, T),
        in_specs=[pl.BlockSpec((None, c4, tile), lambda n, t: (n, 0, t)),
                  _const_spec(w2b), _const_spec(s1), _const_spec(t1)],
        out_specs=(pl.BlockSpec((None, c4, tile), lambda n, t: (n, 0, t)),
                   stats_spec(c4), stats_spec(c4)),
        compiler_params=_params(),
    )(h1, w2b, s1, t1)
    s2, t2 = _bn_fold(s2p, q2p, g2, b2, count)

    a2, s3p, q3p = pl.pallas_call(
        _pass3_kernel,
        out_shape=(jax.ShapeDtypeStruct((N, c4, HW), jnp.bfloat16),
                   stats_sd(Cout), stats_sd(Cout)),
        grid=(N, T),
        in_specs=[pl.BlockSpec((None, c4, tile), lambda n, t: (n, 0, t)),
                  _const_spec(w3b), _const_spec(s2), _const_spec(t2)],
        out_specs=(pl.BlockSpec((None, c4, tile), lambda n, t: (n, 0, t)),
                   stats_spec(Cout), stats_spec(Cout)),
        compiler_params=_params(),
    )(h2, w3b, s2, t2)
    s3, t3 = _bn_fold(s3p, q3p, g3, b3, count)

    out3 = pl.pallas_call(
        _pass4_kernel,
        out_shape=jax.ShapeDtypeStruct((N, Cout, HW), x.dtype),
        grid=(N, T),
        in_specs=[pl.BlockSpec((None, Cin, tile), lambda n, t: (n, 0, t)),
                  pl.BlockSpec((None, c4, tile), lambda n, t: (n, 0, t)),
                  _const_spec(w3b), _const_spec(s3), _const_spec(t3)],
        out_specs=pl.BlockSpec((None, Cout, tile), lambda n, t: (n, 0, t)),
        compiler_params=_params(),
    )(x3, a2, w3b, s3, t3)
    return out3.reshape(N, Cout, H, W)
